# Initial kernel scaffold; baseline (speedup 1.0000x reference)
import jax
import jax.numpy as jnp
from jax.experimental import pallas as pl


def kernel(x, edge_index, edge_attr, batch, embeds, c1_neg_w, c1_neg_b, c1_root_w, c1_root_b, c2_neg_w, c2_neg_b, c2_root_w, c2_root_b, g1_w, g1_b, g2_w, g2_b, g3_w, g3_b, semi_w, bn_g, bn_b, fin_w, fin_b):
    raise NotImplementedError("write your pallas kernel here")


# speedup vs baseline: 1.9654x; 1.9654x over previous
import functools

import jax
import jax.numpy as jnp
from jax import lax
from jax.experimental import pallas as pl
from jax.experimental.pallas import tpu as pltpu
from jax.experimental.pallas import tpu_sc as plsc

N = 50000
E = 800000
G = 256
NC = 2
NS = 16
NW = NC * NS

BE = 3200
BN = 2000
CK = 128
NK = E // CK
CH = 800
NCH = E // CH
RPT = N // NS

def _sc_mesh():
    return plsc.VectorSubcoreMesh(
        core_axis_name="c", subcore_axis_name="s",
        num_cores=NC, num_subcores=NS)


def _wid():
    return lax.axis_index("s") * NC + lax.axis_index("c")


def _sc_gather_x(x1d, src):
    fn = pl.kernel(
        _sc_gather_x_body,
        out_type=jax.ShapeDtypeStruct((E,), jnp.int32),
        mesh=_sc_mesh(),
        compiler_params=pltpu.CompilerParams(
            use_tc_tiling_on_sc=False, needs_layout_passes=False),
        scratch_types=[
            pltpu.VMEM((N,), jnp.int32),
            pltpu.VMEM((CH,), jnp.int32),
            pltpu.VMEM((CH,), jnp.int32),
        ],
    )
    return fn(x1d, src)


def _sc_gather_x_body(x_hbm, src_hbm, out_hbm, xbuf, sbuf, obuf):
    w = _wid()
    pltpu.sync_copy(x_hbm, xbuf)

    def chunk(i, carry):
        k = w + NW * i

        @pl.when(k < NCH)
        def _():
            pltpu.sync_copy(src_hbm.at[pl.ds(k * CH, CH)], sbuf)

            def inner(j, c2):
                idx = sbuf[pl.ds(j * 16, 16)]
                obuf[pl.ds(j * 16, 16)] = plsc.load_gather(xbuf, [idx])
                return c2

            lax.fori_loop(0, CH // 16, inner, 0)
            pltpu.sync_copy(obuf, out_hbm.at[pl.ds(k * CH, CH)])

        return carry

    lax.fori_loop(0, (NCH + NW - 1) // NW, chunk, 0)


def _sc_gather_rows(tab, src):
    fn = pl.kernel(
        _sc_gather_rows_body,
        out_type=jax.ShapeDtypeStruct((E, 16), jnp.float32),
        mesh=_sc_mesh(),
        compiler_params=pltpu.CompilerParams(use_tc_tiling_on_sc=False),
        scratch_types=[
            pltpu.VMEM((CK,), jnp.int32),
            pltpu.VMEM((CK, 16), jnp.float32),
            pltpu.SemaphoreType.DMA,
        ],
    )
    return fn(tab, src)


def _sc_gather_rows_body(tab_hbm, src_hbm, out_hbm, ibuf, rbuf, sem):
    w = _wid()

    def chunk(i, carry):
        k = w + NW * i

        @pl.when(k < NK)
        def _():
            pltpu.sync_copy(src_hbm.at[pl.ds(k * CK, CK)], ibuf)
            pltpu.async_copy(tab_hbm.at[ibuf], rbuf, sem).wait()
            pltpu.sync_copy(rbuf, out_hbm.at[pl.ds(k * CK, CK)])

        return carry

    lax.fori_loop(0, (NK + NW - 1) // NW, chunk, 0)


def _sc_scatter1(msg, dst, root):
    fn = pl.kernel(
        _sc_scatter1_body,
        out_type=jax.ShapeDtypeStruct((NC, N, 32), jnp.float32),
        mesh=_sc_mesh(),
        compiler_params=pltpu.CompilerParams(use_tc_tiling_on_sc=False),
        scratch_types=[
            pltpu.MemorySpace.VMEM_SHARED((N, 32), jnp.float32),
            pltpu.VMEM((CK,), jnp.int32),
            pltpu.VMEM((CK, 32), jnp.float32),
        ],
    )
    return fn(msg, dst, root)


def _sc_scatter1_body(msg_hbm, dst_hbm, root_hbm, out_hbm, acc, ibuf, mbuf):
    c = lax.axis_index("c")
    s = lax.axis_index("s")
    rows = pl.ds(s * RPT, RPT)
    pltpu.sync_copy(root_hbm.at[c, rows], acc.at[rows])
    plsc.subcore_barrier()

    def chunk(i, carry):
        k = s + NS * i

        @pl.when(k < NK)
        def _():
            pltpu.sync_copy(dst_hbm.at[pl.ds(k * CK, CK)], ibuf)
            pltpu.sync_copy(msg_hbm.at[c, pl.ds(k * CK, CK)], mbuf)
            pltpu.sync_copy(mbuf, acc.at[ibuf], add=True)

        return carry

    lax.fori_loop(0, (NK + NS - 1) // NS, chunk, 0)
    plsc.subcore_barrier()
    pltpu.sync_copy(acc.at[rows], out_hbm.at[c, rows])


NK2 = NK // NC


def _sc_scatter2(msg2, dst, init2):
    fn = pl.kernel(
        _sc_scatter2_body,
        out_type=jax.ShapeDtypeStruct((NC, N, 16), jnp.float32),
        mesh=_sc_mesh(),
        compiler_params=pltpu.CompilerParams(use_tc_tiling_on_sc=False),
        scratch_types=[
            pltpu.MemorySpace.VMEM_SHARED((N, 16), jnp.float32),
            pltpu.VMEM((CK,), jnp.int32),
            pltpu.VMEM((CK, 16), jnp.float32),
        ],
    )
    return fn(msg2, dst, init2)


def _sc_scatter2_body(msg_hbm, dst_hbm, init_hbm, out_hbm, acc, ibuf, mbuf):
    c = lax.axis_index("c")
    s = lax.axis_index("s")
    rows = pl.ds(s * RPT, RPT)
    pltpu.sync_copy(init_hbm.at[c, rows], acc.at[rows])
    plsc.subcore_barrier()

    def chunk(i, carry):
        kk = s + NS * i

        @pl.when(kk < NK2)
        def _():
            k = c * NK2 + kk
            pltpu.sync_copy(dst_hbm.at[pl.ds(k * CK, CK)], ibuf)
            pltpu.sync_copy(msg_hbm.at[pl.ds(k * CK, CK)], mbuf)
            pltpu.sync_copy(mbuf, acc.at[ibuf], add=True)

        return carry

    lax.fori_loop(0, (NK2 + NS - 1) // NS, chunk, 0)
    plsc.subcore_barrier()
    pltpu.sync_copy(acc.at[rows], out_hbm.at[c, rows])


def _msg1_body(xsrc_ref, attr_ref, emb_ref, nw_ref, nb_ref, lo_ref, hi_ref):
    t1 = jnp.dot(emb_ref[...], nw_ref[0:64, :],
                 preferred_element_type=jnp.float32) + nb_ref[...]
    xs = xsrc_ref[0, 0, :]
    onehot = (xs[:, None] == lax.broadcasted_iota(jnp.int32, (BE, 24), 1))
    hpart = jnp.dot(onehot.astype(jnp.float32), t1,
                    precision=lax.Precision.HIGHEST,
                    preferred_element_type=jnp.float32)
    apart = lax.dot_general(attr_ref[...], nw_ref[64:70, :],
                            (((1,), (0,)), ((), ())),
                            preferred_element_type=jnp.float32)
    msg = jnp.tanh(hpart + apart)
    lo_ref[...] = msg[:, 0:32]
    hi_ref[...] = msg[:, 32:64]


def _root1_body(x_ref, emb_ref, rw_ref, rb_ref, lo_ref, hi_ref):
    r1 = jnp.tanh(jnp.dot(emb_ref[...], rw_ref[...],
                          preferred_element_type=jnp.float32) + rb_ref[...])
    xs = x_ref[0, 0, :]
    onehot = (xs[:, None] == lax.broadcasted_iota(jnp.int32, (BN, 24), 1))
    root = jnp.dot(onehot.astype(jnp.float32), r1,
                   precision=lax.Precision.HIGHEST,
                   preferred_element_type=jnp.float32)
    lo_ref[...] = root[:, 0:32]
    hi_ref[...] = root[:, 32:64]


def _node2_body(h1a_ref, h1b_ref, nw_ref, rw_ref, rb_ref, p2_ref, r2_ref):
    p2_ref[...] = (
        jnp.dot(h1a_ref[...], nw_ref[0:32, :],
                preferred_element_type=jnp.float32)
        + jnp.dot(h1b_ref[...], nw_ref[32:64, :],
                  preferred_element_type=jnp.float32))
    r2_ref[...] = jnp.tanh(
        jnp.dot(h1a_ref[...], rw_ref[0:32, :],
                preferred_element_type=jnp.float32)
        + jnp.dot(h1b_ref[...], rw_ref[32:64, :],
                  preferred_element_type=jnp.float32)
        + rb_ref[...])


def _msg2_body(p2src_ref, attr_ref, nw_ref, nb_ref, out_ref):
    apart = lax.dot_general(attr_ref[...], nw_ref[64:70, :],
                            (((1,), (0,)), ((), ())),
                            preferred_element_type=jnp.float32)
    out_ref[...] = jnp.tanh(p2src_ref[...] + apart + nb_ref[...])


def _gate1_body(q0_ref, q1_ref, batch_ref, g1w_ref, g1b_ref, g2w_ref,
                g2b_ref, g3w_ref, g3b_ref, gate_ref, h2_ref, smax_ref,
                gsum_ref):
    i = pl.program_id(0)
    h2 = q0_ref[0] + q1_ref[0]
    g = jnp.maximum(jnp.dot(h2, g1w_ref[...],
                            preferred_element_type=jnp.float32)
                    + g1b_ref[...], 0.0)
    g = jnp.maximum(jnp.dot(g, g2w_ref[...],
                            preferred_element_type=jnp.float32)
                    + g2b_ref[...], 0.0)
    gate = jnp.dot(g, g3w_ref[...],
                   preferred_element_type=jnp.float32) + g3b_ref[...]
    gate_ref[...] = gate
    h2_ref[...] = h2

    b = batch_ref[0, 0, :]
    onehot = b[:, None] == lax.broadcasted_iota(jnp.int32, (BN, G), 1)
    masked = jnp.where(onehot, gate, -3.4e38)
    blockmax = jnp.max(masked, axis=0, keepdims=True)

    @pl.when(i == 0)
    def _():
        smax_ref[...] = jnp.full((1, G), -3.4e38, jnp.float32)
        gsum_ref[...] = jnp.zeros((1, 128), jnp.float32)

    smax_ref[...] = jnp.maximum(smax_ref[...], blockmax)
    gsum_ref[...] = gsum_ref[...] + jnp.broadcast_to(jnp.sum(gate), (1, 128))


def _pool_body(gate_ref, h2_ref, batch_ref, smax_ref, den_ref, num_ref):
    i = pl.program_id(0)
    b = batch_ref[0, 0, :]
    onehot = (b[:, None] == lax.broadcasted_iota(jnp.int32, (BN, G), 1)
              ).astype(jnp.float32)
    smx = lax.dot_general(onehot, smax_ref[...], (((1,), (1,)), ((), ())),
                          precision=lax.Precision.HIGHEST,
                          preferred_element_type=jnp.float32)
    ge = jnp.exp(gate_ref[...] - smx)

    @pl.when(i == 0)
    def _():
        den_ref[...] = jnp.zeros((1, G), jnp.float32)
        num_ref[...] = jnp.zeros((16, G), jnp.float32)

    den_ref[...] = den_ref[...] + lax.dot_general(
        ge, onehot, (((0,), (0,)), ((), ())),
        precision=lax.Precision.HIGHEST,
        preferred_element_type=jnp.float32)
    num_ref[...] = num_ref[...] + lax.dot_general(
        ge * h2_ref[...], onehot, (((0,), (0,)), ((), ())),
        precision=lax.Precision.HIGHEST,
        preferred_element_type=jnp.float32)


def _att_body(gate_ref, gsum_ref, att_ref):
    att_ref[...] = gate_ref[...] - gsum_ref[0, 0] / N


def _head_body(num_ref, den_ref, sw_ref, bg_ref, bb_ref, fw_ref, fb_ref,
               o1_ref, out_ref, sig_ref):
    emb = num_ref[...] / (den_ref[...] + 1e-16)
    o1 = lax.dot_general(emb, sw_ref[...], (((0,), (0,)), ((), ())),
                         preferred_element_type=jnp.float32)
    mu = jnp.mean(o1, axis=0, keepdims=True)
    var = jnp.mean((o1 - mu) * (o1 - mu), axis=0, keepdims=True)
    o1n = (o1 - mu) / jnp.sqrt(var + 1e-5) * bg_ref[...] + bb_ref[...]
    o1_ref[...] = o1n
    out = lax.dot_general(fw_ref[...], o1n, (((0,), (1,)), ((), ())),
                          preferred_element_type=jnp.float32) + fb_ref[...]
    out_ref[...] = out
    sig_ref[...] = jax.nn.sigmoid(out)


def _full(shape):
    return pl.BlockSpec(shape, lambda *_: tuple(0 for _ in shape))


def kernel(x, edge_index, edge_attr, batch, embeds, c1_neg_w, c1_neg_b,
           c1_root_w, c1_root_b, c2_neg_w, c2_neg_b, c2_root_w, c2_root_b,
           g1_w, g1_b, g2_w, g2_b, g3_w, g3_b, semi_w, bn_g, bn_b,
           fin_w, fin_b):
    x = x.astype(jnp.int32)
    dst = edge_index[0].astype(jnp.int32)
    src = edge_index[1].astype(jnp.int32)

    xsrc = _sc_gather_x(x, src)

    msg_lo, msg_hi = pl.pallas_call(
        _msg1_body,
        grid=(E // BE,),
        in_specs=[
            pl.BlockSpec((1, 1, BE), lambda i: (i, 0, 0)),
            pl.BlockSpec((BE, 6), lambda i: (i, 0)),
            _full((24, 64)), _full((70, 64)), _full((1, 64)),
        ],
        out_specs=[pl.BlockSpec((BE, 32), lambda i: (i, 0))] * 2,
        out_shape=[jax.ShapeDtypeStruct((E, 32), jnp.float32)] * 2,
    )(xsrc.reshape(E // BE, 1, BE), edge_attr, embeds, c1_neg_w,
      c1_neg_b.reshape(1, 64))

    root_lo, root_hi = pl.pallas_call(
        _root1_body,
        grid=(N // BN,),
        in_specs=[
            pl.BlockSpec((1, 1, BN), lambda i: (i, 0, 0)),
            _full((24, 64)), _full((64, 64)), _full((1, 64)),
        ],
        out_specs=[pl.BlockSpec((BN, 32), lambda i: (i, 0))] * 2,
        out_shape=[jax.ShapeDtypeStruct((N, 32), jnp.float32)] * 2,
    )(x.reshape(N // BN, 1, BN), embeds, c1_root_w, c1_root_b.reshape(1, 64))

    msg = jnp.stack([msg_lo, msg_hi])
    root = jnp.stack([root_lo, root_hi])
    h1 = _sc_scatter1(msg, dst, root)
    h1a, h1b = h1[0], h1[1]

    p2, r2 = pl.pallas_call(
        _node2_body,
        grid=(N // BN,),
        in_specs=[
            pl.BlockSpec((BN, 32), lambda i: (i, 0)),
            pl.BlockSpec((BN, 32), lambda i: (i, 0)),
            _full((70, 16)), _full((64, 16)), _full((1, 16)),
        ],
        out_specs=[pl.BlockSpec((BN, 16), lambda i: (i, 0))] * 2,
        out_shape=[jax.ShapeDtypeStruct((N, 16), jnp.float32)] * 2,
    )(h1a, h1b, c2_neg_w, c2_root_w, c2_root_b.reshape(1, 16))

    p2src = _sc_gather_rows(p2, src)

    msg2 = pl.pallas_call(
        _msg2_body,
        grid=(E // BE,),
        in_specs=[
            pl.BlockSpec((BE, 16), lambda i: (i, 0)),
            pl.BlockSpec((BE, 6), lambda i: (i, 0)),
            _full((70, 16)), _full((1, 16)),
        ],
        out_specs=pl.BlockSpec((BE, 16), lambda i: (i, 0)),
        out_shape=jax.ShapeDtypeStruct((E, 16), jnp.float32),
    )(p2src, edge_attr, c2_neg_w, c2_neg_b.reshape(1, 16))

    init2 = jnp.stack([r2, jnp.zeros_like(r2)])
    q = _sc_scatter2(msg2, dst, init2)

    gate, h2, smax, gsum = pl.pallas_call(
        _gate1_body,
        grid=(N // BN,),
        in_specs=[
            pl.BlockSpec((1, BN, 16), lambda i: (0, i, 0)),
            pl.BlockSpec((1, BN, 16), lambda i: (1, i, 0)),
            pl.BlockSpec((1, 1, BN), lambda i: (i, 0, 0)),
            _full((16, 64)), _full((1, 64)), _full((64, 32)), _full((1, 32)),
            _full((32, 1)), _full((1, 1)),
        ],
        out_specs=[
            pl.BlockSpec((BN, 1), lambda i: (i, 0)),
            pl.BlockSpec((BN, 16), lambda i: (i, 0)),
            _full((1, G)), _full((1, 128)),
        ],
        out_shape=[
            jax.ShapeDtypeStruct((N, 1), jnp.float32),
            jax.ShapeDtypeStruct((N, 16), jnp.float32),
            jax.ShapeDtypeStruct((1, G), jnp.float32),
            jax.ShapeDtypeStruct((1, 128), jnp.float32),
        ],
    )(q, q, batch.reshape(N // BN, 1, BN).astype(jnp.int32),
      g1_w, g1_b.reshape(1, 64), g2_w, g2_b.reshape(1, 32),
      g3_w, g3_b.reshape(1, 1))

    den, num = pl.pallas_call(
        _pool_body,
        grid=(N // BN,),
        in_specs=[
            pl.BlockSpec((BN, 1), lambda i: (i, 0)),
            pl.BlockSpec((BN, 16), lambda i: (i, 0)),
            pl.BlockSpec((1, 1, BN), lambda i: (i, 0, 0)),
            _full((1, G)),
        ],
        out_specs=[_full((1, G)), _full((16, G))],
        out_shape=[
            jax.ShapeDtypeStruct((1, G), jnp.float32),
            jax.ShapeDtypeStruct((16, G), jnp.float32),
        ],
    )(gate, h2, batch.reshape(N // BN, 1, BN).astype(jnp.int32), smax)

    att = pl.pallas_call(
        _att_body,
        grid=(N // BN,),
        in_specs=[pl.BlockSpec((BN, 1), lambda i: (i, 0)), _full((1, 128))],
        out_specs=pl.BlockSpec((BN, 1), lambda i: (i, 0)),
        out_shape=jax.ShapeDtypeStruct((N, 1), jnp.float32),
    )(gate, gsum)

    o1, outv, sig = pl.pallas_call(
        _head_body,
        in_specs=[
            _full((16, G)), _full((1, G)), _full((16, 200)),
            _full((1, 200)), _full((1, 200)), _full((200, 1)), _full((1, 1)),
        ],
        out_specs=[_full((G, 200)), _full((1, G)), _full((1, G))],
        out_shape=[
            jax.ShapeDtypeStruct((G, 200), jnp.float32),
            jax.ShapeDtypeStruct((1, G), jnp.float32),
            jax.ShapeDtypeStruct((1, G), jnp.float32),
        ],
    )(num, den, semi_w, bn_g.reshape(1, 200), bn_b.reshape(1, 200),
      fin_w, fin_b.reshape(1, 1))

    return outv.reshape(-1), sig.reshape(-1), att, o1


# speedup vs baseline: 2.8807x; 1.4657x over previous
import functools

import jax
import jax.numpy as jnp
from jax import lax
from jax.experimental import pallas as pl
from jax.experimental.pallas import tpu as pltpu
from jax.experimental.pallas import tpu_sc as plsc

N = 50000
E = 800000
G = 256
NC = 2
NS = 16
NW = NC * NS

BE = 8000
BN = 2000
CH = 800
NCH = E // CH
RPT = N // NS
CKS = 125
SUB = 5
BK = CKS * SUB

def _sc_mesh():
    return plsc.VectorSubcoreMesh(
        core_axis_name="c", subcore_axis_name="s",
        num_cores=NC, num_subcores=NS)


def _wid():
    return lax.axis_index("s") * NC + lax.axis_index("c")


def _sc_gather_x(x1d, src):
    fn = pl.kernel(
        _sc_gather_x_body,
        out_type=jax.ShapeDtypeStruct((E,), jnp.int32),
        mesh=_sc_mesh(),
        compiler_params=pltpu.CompilerParams(
            use_tc_tiling_on_sc=False, needs_layout_passes=False),
        scratch_types=[
            pltpu.VMEM((N,), jnp.int32),
            pltpu.VMEM((CH,), jnp.int32),
            pltpu.VMEM((CH,), jnp.int32),
        ],
    )
    return fn(x1d, src)


def _sc_gather_x_body(x_hbm, src_hbm, out_hbm, xbuf, sbuf, obuf):
    w = _wid()
    pltpu.sync_copy(x_hbm, xbuf)

    def chunk(i, carry):
        k = w + NW * i

        @pl.when(k < NCH)
        def _():
            pltpu.sync_copy(src_hbm.at[pl.ds(k * CH, CH)], sbuf)

            def inner(j, c2):
                idx = sbuf[pl.ds(j * 16, 16)]
                obuf[pl.ds(j * 16, 16)] = plsc.load_gather(xbuf, [idx])
                return c2

            lax.fori_loop(0, CH // 16, inner, 0)
            pltpu.sync_copy(obuf, out_hbm.at[pl.ds(k * CH, CH)])

        return carry

    lax.fori_loop(0, (NCH + NW - 1) // NW, chunk, 0)


def _sc_gather_rows(tab, src2d):
    fn = pl.kernel(
        _sc_gather_rows_body,
        out_type=jax.ShapeDtypeStruct((E, 16), jnp.float32),
        mesh=_sc_mesh(),
        compiler_params=pltpu.CompilerParams(use_tc_tiling_on_sc=False),
        scratch_types=[
            pltpu.VMEM((SUB, CKS), jnp.int32),
            pltpu.VMEM((BK, 16), jnp.float32),
            pltpu.SemaphoreType.DMA,
        ],
    )
    return fn(tab, src2d)


def _sc_gather_rows_body(tab_hbm, src_hbm, out_hbm, ibuf, rbuf, sem):
    w = _wid()
    epw = E // NW

    def it(i, carry):
        pltpu.sync_copy(src_hbm.at[pl.ds(w * (epw // CKS) + i * SUB, SUB)],
                        ibuf)
        descs = [
            pltpu.async_copy(tab_hbm.at[ibuf.at[j]],
                             rbuf.at[pl.ds(j * CKS, CKS)], sem)
            for j in range(SUB)
        ]
        for d in descs:
            d.wait()
        pltpu.sync_copy(rbuf, out_hbm.at[pl.ds(w * epw + i * BK, BK)])
        return carry

    lax.fori_loop(0, epw // BK, it, 0)


def _sc_scatter1(msg, dst2d, root):
    fn = pl.kernel(
        _sc_scatter1_body,
        out_type=jax.ShapeDtypeStruct((NC, N, 32), jnp.float32),
        mesh=_sc_mesh(),
        compiler_params=pltpu.CompilerParams(use_tc_tiling_on_sc=False),
        scratch_types=[
            pltpu.MemorySpace.VMEM_SHARED((N, 32), jnp.float32),
            pltpu.VMEM((SUB, CKS), jnp.int32),
            pltpu.VMEM((BK, 32), jnp.float32),
            pltpu.SemaphoreType.DMA,
        ],
    )
    return fn(msg, dst2d, root)


def _sc_scatter1_body(msg_hbm, dst_hbm, root_hbm, out_hbm, acc, ibuf, mbuf,
                      sem):
    c = lax.axis_index("c")
    s = lax.axis_index("s")
    rows = pl.ds(s * RPT, RPT)
    pltpu.sync_copy(root_hbm.at[c, rows], acc.at[rows])
    plsc.subcore_barrier()
    epw = E // NS

    def it(i, carry):
        pltpu.sync_copy(dst_hbm.at[pl.ds(s * (epw // CKS) + i * SUB, SUB)],
                        ibuf)
        pltpu.sync_copy(msg_hbm.at[c, pl.ds(s * epw + i * BK, BK)], mbuf)
        descs = [
            pltpu.async_copy(mbuf.at[pl.ds(j * CKS, CKS)],
                             acc.at[ibuf.at[j]], sem, add=True)
            for j in range(SUB)
        ]
        for d in descs:
            d.wait()
        return carry

    lax.fori_loop(0, epw // BK, it, 0)
    plsc.subcore_barrier()
    pltpu.sync_copy(acc.at[rows], out_hbm.at[c, rows])


def _sc_scatter2(msg2, dst2d, init2):
    fn = pl.kernel(
        _sc_scatter2_body,
        out_type=jax.ShapeDtypeStruct((NC, N, 16), jnp.float32),
        mesh=_sc_mesh(),
        compiler_params=pltpu.CompilerParams(use_tc_tiling_on_sc=False),
        scratch_types=[
            pltpu.MemorySpace.VMEM_SHARED((N, 16), jnp.float32),
            pltpu.VMEM((SUB, CKS), jnp.int32),
            pltpu.VMEM((BK, 16), jnp.float32),
            pltpu.SemaphoreType.DMA,
        ],
    )
    return fn(msg2, dst2d, init2)


def _sc_scatter2_body(msg_hbm, dst_hbm, init_hbm, out_hbm, acc, ibuf, mbuf,
                      sem):
    c = lax.axis_index("c")
    s = lax.axis_index("s")
    rows = pl.ds(s * RPT, RPT)
    pltpu.sync_copy(init_hbm.at[c, rows], acc.at[rows])
    plsc.subcore_barrier()
    epw = E // NC // NS

    def it(i, carry):
        e0 = c * (E // NC) + s * epw + i * BK
        pltpu.sync_copy(dst_hbm.at[pl.ds(e0 // CKS, SUB)], ibuf)
        pltpu.sync_copy(msg_hbm.at[pl.ds(e0, BK)], mbuf)
        descs = [
            pltpu.async_copy(mbuf.at[pl.ds(j * CKS, CKS)],
                             acc.at[ibuf.at[j]], sem, add=True)
            for j in range(SUB)
        ]
        for d in descs:
            d.wait()
        return carry

    lax.fori_loop(0, epw // BK, it, 0)
    plsc.subcore_barrier()
    pltpu.sync_copy(acc.at[rows], out_hbm.at[c, rows])


def _msg1_body(xsrc_ref, attr_ref, emb_ref, nw_ref, nb_ref, out_ref):
    t1 = jnp.dot(emb_ref[...], nw_ref[0:64, :],
                 preferred_element_type=jnp.float32) + nb_ref[...]
    xs = xsrc_ref[0, 0, :]
    onehot = (xs[:, None] == lax.broadcasted_iota(jnp.int32, (BE, 24), 1))
    hpart = jnp.dot(onehot.astype(jnp.float32), t1,
                    precision=lax.Precision.HIGHEST,
                    preferred_element_type=jnp.float32)
    apart = lax.dot_general(attr_ref[...], nw_ref[64:70, :],
                            (((1,), (0,)), ((), ())),
                            preferred_element_type=jnp.float32)
    msg = jnp.tanh(hpart + apart)
    out_ref[0] = msg[:, 0:32]
    out_ref[1] = msg[:, 32:64]


def _root1_body(x_ref, emb_ref, rw_ref, rb_ref, out_ref):
    r1 = jnp.tanh(jnp.dot(emb_ref[...], rw_ref[...],
                          preferred_element_type=jnp.float32) + rb_ref[...])
    xs = x_ref[0, 0, :]
    onehot = (xs[:, None] == lax.broadcasted_iota(jnp.int32, (BN, 24), 1))
    root = jnp.dot(onehot.astype(jnp.float32), r1,
                   precision=lax.Precision.HIGHEST,
                   preferred_element_type=jnp.float32)
    out_ref[0] = root[:, 0:32]
    out_ref[1] = root[:, 32:64]


def _node2_body(h1a_ref, h1b_ref, nw_ref, rw_ref, rb_ref, p2_ref, init_ref):
    p2_ref[...] = (
        jnp.dot(h1a_ref[...], nw_ref[0:32, :],
                preferred_element_type=jnp.float32)
        + jnp.dot(h1b_ref[...], nw_ref[32:64, :],
                  preferred_element_type=jnp.float32))
    init_ref[0] = jnp.tanh(
        jnp.dot(h1a_ref[...], rw_ref[0:32, :],
                preferred_element_type=jnp.float32)
        + jnp.dot(h1b_ref[...], rw_ref[32:64, :],
                  preferred_element_type=jnp.float32)
        + rb_ref[...])
    init_ref[1] = jnp.zeros((BN, 16), jnp.float32)


def _msg2_body(p2src_ref, attr_ref, nw_ref, nb_ref, out_ref):
    apart = lax.dot_general(attr_ref[...], nw_ref[64:70, :],
                            (((1,), (0,)), ((), ())),
                            preferred_element_type=jnp.float32)
    out_ref[...] = jnp.tanh(p2src_ref[...] + apart + nb_ref[...])


def _gate1_body(q0_ref, q1_ref, batch_ref, g1w_ref, g1b_ref, g2w_ref,
                g2b_ref, g3w_ref, g3b_ref, gate_ref, h2_ref, smax_ref,
                gsum_ref):
    i = pl.program_id(0)
    h2 = q0_ref[0] + q1_ref[0]
    g = jnp.maximum(jnp.dot(h2, g1w_ref[...],
                            preferred_element_type=jnp.float32)
                    + g1b_ref[...], 0.0)
    g = jnp.maximum(jnp.dot(g, g2w_ref[...],
                            preferred_element_type=jnp.float32)
                    + g2b_ref[...], 0.0)
    gate = jnp.dot(g, g3w_ref[...],
                   preferred_element_type=jnp.float32) + g3b_ref[...]
    gate_ref[...] = gate
    h2_ref[...] = h2

    b = batch_ref[0, 0, :]
    onehot = b[:, None] == lax.broadcasted_iota(jnp.int32, (BN, G), 1)
    masked = jnp.where(onehot, gate, -3.4e38)
    blockmax = jnp.max(masked, axis=0, keepdims=True)

    @pl.when(i == 0)
    def _():
        smax_ref[...] = jnp.full((1, G), -3.4e38, jnp.float32)
        gsum_ref[...] = jnp.zeros((1, 128), jnp.float32)

    smax_ref[...] = jnp.maximum(smax_ref[...], blockmax)
    gsum_ref[...] = gsum_ref[...] + jnp.broadcast_to(jnp.sum(gate), (1, 128))


def _pool_body(gate_ref, h2_ref, batch_ref, smax_ref, gsum_ref,
               den_ref, num_ref, att_ref):
    i = pl.program_id(0)
    b = batch_ref[0, 0, :]
    onehot = (b[:, None] == lax.broadcasted_iota(jnp.int32, (BN, G), 1)
              ).astype(jnp.float32)
    smx = lax.dot_general(onehot, smax_ref[...], (((1,), (1,)), ((), ())),
                          precision=lax.Precision.HIGHEST,
                          preferred_element_type=jnp.float32)
    ge = jnp.exp(gate_ref[...] - smx)

    @pl.when(i == 0)
    def _():
        den_ref[...] = jnp.zeros((1, G), jnp.float32)
        num_ref[...] = jnp.zeros((16, G), jnp.float32)

    den_ref[...] = den_ref[...] + lax.dot_general(
        ge, onehot, (((0,), (0,)), ((), ())),
        precision=lax.Precision.HIGHEST,
        preferred_element_type=jnp.float32)
    num_ref[...] = num_ref[...] + lax.dot_general(
        ge * h2_ref[...], onehot, (((0,), (0,)), ((), ())),
        precision=lax.Precision.HIGHEST,
        preferred_element_type=jnp.float32)
    att_ref[...] = gate_ref[...] - gsum_ref[0, 0] / N


def _head_body(num_ref, den_ref, sw_ref, bg_ref, bb_ref, fw_ref, fb_ref,
               o1_ref, out_ref, sig_ref):
    emb = num_ref[...] / (den_ref[...] + 1e-16)
    o1 = lax.dot_general(emb, sw_ref[...], (((0,), (0,)), ((), ())),
                         preferred_element_type=jnp.float32)
    mu = jnp.mean(o1, axis=0, keepdims=True)
    var = jnp.mean((o1 - mu) * (o1 - mu), axis=0, keepdims=True)
    o1n = (o1 - mu) / jnp.sqrt(var + 1e-5) * bg_ref[...] + bb_ref[...]
    o1_ref[...] = o1n
    out = lax.dot_general(fw_ref[...], o1n, (((0,), (1,)), ((), ())),
                          preferred_element_type=jnp.float32) + fb_ref[...]
    out_ref[...] = out
    sig_ref[...] = jax.nn.sigmoid(out)


def _full(shape):
    return pl.BlockSpec(shape, lambda *_: tuple(0 for _ in shape))


def kernel(x, edge_index, edge_attr, batch, embeds, c1_neg_w, c1_neg_b,
           c1_root_w, c1_root_b, c2_neg_w, c2_neg_b, c2_root_w, c2_root_b,
           g1_w, g1_b, g2_w, g2_b, g3_w, g3_b, semi_w, bn_g, bn_b,
           fin_w, fin_b):
    x = x.astype(jnp.int32)
    dst = edge_index[0].astype(jnp.int32)
    src = edge_index[1].astype(jnp.int32)
    dst2d = dst.reshape(E // CKS, CKS)
    src2d = src.reshape(E // CKS, CKS)

    xsrc = _sc_gather_x(x, src)

    msg = pl.pallas_call(
        _msg1_body,
        grid=(E // BE,),
        in_specs=[
            pl.BlockSpec((1, 1, BE), lambda i: (i, 0, 0)),
            pl.BlockSpec((BE, 6), lambda i: (i, 0)),
            _full((24, 64)), _full((70, 64)), _full((1, 64)),
        ],
        out_specs=pl.BlockSpec((2, BE, 32), lambda i: (0, i, 0)),
        out_shape=jax.ShapeDtypeStruct((2, E, 32), jnp.float32),
    )(xsrc.reshape(E // BE, 1, BE), edge_attr, embeds, c1_neg_w,
      c1_neg_b.reshape(1, 64))

    root = pl.pallas_call(
        _root1_body,
        grid=(N // BN,),
        in_specs=[
            pl.BlockSpec((1, 1, BN), lambda i: (i, 0, 0)),
            _full((24, 64)), _full((64, 64)), _full((1, 64)),
        ],
        out_specs=pl.BlockSpec((2, BN, 32), lambda i: (0, i, 0)),
        out_shape=jax.ShapeDtypeStruct((2, N, 32), jnp.float32),
    )(x.reshape(N // BN, 1, BN), embeds, c1_root_w, c1_root_b.reshape(1, 64))

    h1 = _sc_scatter1(msg, dst2d, root)
    h1a, h1b = h1[0], h1[1]

    p2, init2 = pl.pallas_call(
        _node2_body,
        grid=(N // BN,),
        in_specs=[
            pl.BlockSpec((BN, 32), lambda i: (i, 0)),
            pl.BlockSpec((BN, 32), lambda i: (i, 0)),
            _full((70, 16)), _full((64, 16)), _full((1, 16)),
        ],
        out_specs=[
            pl.BlockSpec((BN, 16), lambda i: (i, 0)),
            pl.BlockSpec((2, BN, 16), lambda i: (0, i, 0)),
        ],
        out_shape=[
            jax.ShapeDtypeStruct((N, 16), jnp.float32),
            jax.ShapeDtypeStruct((2, N, 16), jnp.float32),
        ],
    )(h1a, h1b, c2_neg_w, c2_root_w, c2_root_b.reshape(1, 16))

    p2src = _sc_gather_rows(p2, src2d)

    msg2 = pl.pallas_call(
        _msg2_body,
        grid=(E // BE,),
        in_specs=[
            pl.BlockSpec((BE, 16), lambda i: (i, 0)),
            pl.BlockSpec((BE, 6), lambda i: (i, 0)),
            _full((70, 16)), _full((1, 16)),
        ],
        out_specs=pl.BlockSpec((BE, 16), lambda i: (i, 0)),
        out_shape=jax.ShapeDtypeStruct((E, 16), jnp.float32),
    )(p2src, edge_attr, c2_neg_w, c2_neg_b.reshape(1, 16))

    q = _sc_scatter2(msg2, dst2d, init2)

    gate, h2, smax, gsum = pl.pallas_call(
        _gate1_body,
        grid=(N // BN,),
        in_specs=[
            pl.BlockSpec((1, BN, 16), lambda i: (0, i, 0)),
            pl.BlockSpec((1, BN, 16), lambda i: (1, i, 0)),
            pl.BlockSpec((1, 1, BN), lambda i: (i, 0, 0)),
            _full((16, 64)), _full((1, 64)), _full((64, 32)), _full((1, 32)),
            _full((32, 1)), _full((1, 1)),
        ],
        out_specs=[
            pl.BlockSpec((BN, 1), lambda i: (i, 0)),
            pl.BlockSpec((BN, 16), lambda i: (i, 0)),
            _full((1, G)), _full((1, 128)),
        ],
        out_shape=[
            jax.ShapeDtypeStruct((N, 1), jnp.float32),
            jax.ShapeDtypeStruct((N, 16), jnp.float32),
            jax.ShapeDtypeStruct((1, G), jnp.float32),
            jax.ShapeDtypeStruct((1, 128), jnp.float32),
        ],
    )(q, q, batch.reshape(N // BN, 1, BN).astype(jnp.int32),
      g1_w, g1_b.reshape(1, 64), g2_w, g2_b.reshape(1, 32),
      g3_w, g3_b.reshape(1, 1))

    den, num, att = pl.pallas_call(
        _pool_body,
        grid=(N // BN,),
        in_specs=[
            pl.BlockSpec((BN, 1), lambda i: (i, 0)),
            pl.BlockSpec((BN, 16), lambda i: (i, 0)),
            pl.BlockSpec((1, 1, BN), lambda i: (i, 0, 0)),
            _full((1, G)), _full((1, 128)),
        ],
        out_specs=[
            _full((1, G)), _full((16, G)),
            pl.BlockSpec((BN, 1), lambda i: (i, 0)),
        ],
        out_shape=[
            jax.ShapeDtypeStruct((1, G), jnp.float32),
            jax.ShapeDtypeStruct((16, G), jnp.float32),
            jax.ShapeDtypeStruct((N, 1), jnp.float32),
        ],
    )(gate, h2, batch.reshape(N // BN, 1, BN).astype(jnp.int32), smax, gsum)

    o1, outv, sig = pl.pallas_call(
        _head_body,
        in_specs=[
            _full((16, G)), _full((1, G)), _full((16, 200)),
            _full((1, 200)), _full((1, 200)), _full((200, 1)), _full((1, 1)),
        ],
        out_specs=[_full((G, 200)), _full((1, G)), _full((1, G))],
        out_shape=[
            jax.ShapeDtypeStruct((G, 200), jnp.float32),
            jax.ShapeDtypeStruct((1, G), jnp.float32),
            jax.ShapeDtypeStruct((1, G), jnp.float32),
        ],
    )(num, den, semi_w, bn_g.reshape(1, 200), bn_b.reshape(1, 200),
      fin_w, fin_b.reshape(1, 1))

    return outv.reshape(-1), sig.reshape(-1), att, o1
